# baseline (speedup 1.0000x reference)
import functools

import jax
import jax.numpy as jnp
from jax import lax
from jax.experimental import pallas as pl
from jax.experimental.pallas import tpu as pltpu
from jax.experimental.pallas import tpu_sc as plsc

_LANES = 16
_IDXW = 128
_E = 320000
_N = 10000
_IDX_ROWS = _E // _IDXW
N_PAD = 16 * 626
_P1_BASE = _IDX_ROWS // 16
_P1_REM = _IDX_ROWS - 16 * _P1_BASE
_P2_BASE = _IDX_ROWS // 32
_P2_REM = _IDX_ROWS - 32 * _P2_BASE
_P1_CHUNK = 26
_P2_CHUNK = 26


@functools.partial(
    pl.kernel,
    out_type=jax.ShapeDtypeStruct((2, _IDX_ROWS, 8, _IDXW), jnp.float32),
    mesh=plsc.VectorSubcoreMesh(
        core_axis_name="c", subcore_axis_name="s", num_cores=2, num_subcores=16
    ),
    scratch_types=[
        pltpu.VMEM_SHARED((N_PAD, _LANES), jnp.float32),
        pltpu.VMEM((2, _P1_CHUNK, 8, _IDXW), jnp.float32),
        pltpu.VMEM((_P1_CHUNK * _IDXW, _LANES), jnp.float32),
        pltpu.VMEM((_P1_CHUNK, 2, _IDXW), jnp.int32),
    ],
    compiler_params=pltpu.CompilerParams(
        use_tc_tiling_on_sc=False, needs_layout_passes=False
    ),
)
def _sc_dmpnn(efeat_hbm, eidx_hbm, out_hbm, neigh, wbuf, tbuf, idx_v):
    c = lax.axis_index("c")
    s = lax.axis_index("s")
    iota = lax.iota(jnp.int32, _LANES)
    d0 = iota >> 3
    d2 = iota & 7

    zrows = N_PAD // 16

    def _zero(i, carry):
        tbuf[i] = jnp.zeros((_LANES,), jnp.float32)
        return carry

    lax.fori_loop(0, zrows, _zero, 0)
    pltpu.sync_copy(tbuf.at[pl.ds(0, zrows)], neigh.at[pl.ds(s * zrows, zrows)])
    plsc.subcore_barrier()

    def _load_chunk(rbase, n_idx_rows):
        pltpu.sync_copy(
            eidx_hbm.at[pl.ds(rbase, n_idx_rows)], idx_v.at[pl.ds(0, n_idx_rows)]
        )
        pltpu.sync_copy(
            efeat_hbm.at[0, pl.ds(rbase, n_idx_rows)],
            wbuf.at[0, pl.ds(0, n_idx_rows)],
        )
        pltpu.sync_copy(
            efeat_hbm.at[1, pl.ds(rbase, n_idx_rows)],
            wbuf.at[1, pl.ds(0, n_idx_rows)],
        )

    def _p1_chunk(rbase, n_idx_rows):
        _load_chunk(rbase, n_idx_rows)

        def _repack(e, carry):
            dj = jnp.full((_LANES,), e >> 7, jnp.int32)
            d3 = jnp.full((_LANES,), e & (_IDXW - 1), jnp.int32)
            tbuf[e] = plsc.load_gather(wbuf, [d0, dj, d2, d3])
            return carry

        lax.fori_loop(0, n_idx_rows * _IDXW, _repack, 0)
        for j in range(n_idx_rows):
            pltpu.sync_copy(
                tbuf.at[pl.ds(j * _IDXW, _IDXW)],
                neigh.at[idx_v.at[j, 1]],
                add=True,
            )

    for chunk in range(_P1_BASE // _P1_CHUNK):
        _p1_chunk(s * _P1_BASE + chunk * _P1_CHUNK, _P1_CHUNK)

    @pl.when(s < _P1_REM)
    def _p1_rem():
        _p1_chunk(16 * _P1_BASE + s, 1)

    plsc.subcore_barrier()

    wid = c * 16 + s

    def _p2_chunk(rbase, n_idx_rows):
        pltpu.sync_copy(
            eidx_hbm.at[pl.ds(rbase, n_idx_rows)], idx_v.at[pl.ds(0, n_idx_rows)]
        )
        for j in range(n_idx_rows):
            pltpu.sync_copy(
                neigh.at[idx_v.at[j, 0]], tbuf.at[pl.ds(j * _IDXW, _IDXW)]
            )
        pltpu.sync_copy(
            efeat_hbm.at[0, pl.ds(rbase, n_idx_rows)],
            wbuf.at[0, pl.ds(0, n_idx_rows)],
        )
        pltpu.sync_copy(
            efeat_hbm.at[1, pl.ds(rbase, n_idx_rows)],
            wbuf.at[1, pl.ds(0, n_idx_rows)],
        )

        def _sub(p, carry):
            e = 2 * p
            dj = jnp.full((_LANES,), e >> 7, jnp.int32)
            ei = e & (_IDXW - 1)
            d3e = jnp.full((_LANES,), ei, jnp.int32)
            d3o = d3e + 1
            w_e = plsc.load_gather(wbuf, [d0, dj, d2, d3e])
            w_o = plsc.load_gather(wbuf, [d0, dj, d2, d3o])
            r_e = tbuf[e] - w_o
            r_o = tbuf[e + 1] - w_e
            plsc.store_scatter(wbuf, [d0, dj, d2, d3e], r_e)
            plsc.store_scatter(wbuf, [d0, dj, d2, d3o], r_o)
            return carry

        lax.fori_loop(0, n_idx_rows * _IDXW // 2, _sub, 0)
        pltpu.sync_copy(
            wbuf.at[0, pl.ds(0, n_idx_rows)],
            out_hbm.at[0, pl.ds(rbase, n_idx_rows)],
        )
        pltpu.sync_copy(
            wbuf.at[1, pl.ds(0, n_idx_rows)],
            out_hbm.at[1, pl.ds(rbase, n_idx_rows)],
        )

    for chunk in range(_P2_BASE // _P2_CHUNK):
        _p2_chunk(wid * _P2_BASE + chunk * _P2_CHUNK, _P2_CHUNK)

    @pl.when(wid < _P2_REM)
    def _p2_rem():
        _p2_chunk(32 * _P2_BASE + wid, 1)


def kernel(nfeat, efeat, edge_index):
    efeat_phys = (
        efeat.T.reshape(2, 8, _IDX_ROWS, _IDXW).transpose(0, 2, 1, 3)
    )
    eidx_phys = edge_index.reshape(2, _IDX_ROWS, _IDXW).transpose(1, 0, 2)
    out_phys = _sc_dmpnn(efeat_phys, eidx_phys)
    return (
        out_phys.transpose(0, 2, 1, 3).reshape(_LANES, _E).T
    )


# speedup vs baseline: 1.0872x; 1.0872x over previous
import functools

import jax
import jax.numpy as jnp
from jax import lax
from jax.experimental import pallas as pl
from jax.experimental.pallas import tpu as pltpu
from jax.experimental.pallas import tpu_sc as plsc

_LANES = 16
_IDXW = 128
_E = 320000
_N = 10000
_IDX_ROWS = _E // _IDXW
N_PAD = 16 * 626
_P1_BASE = _IDX_ROWS // 16
_P1_REM = _IDX_ROWS - 16 * _P1_BASE
_P2_BASE = _IDX_ROWS // 32
_P2_REM = _IDX_ROWS - 32 * _P2_BASE
_CH = 13
_P1_NCH = _P1_BASE // _CH
_P2_NCH = _P2_BASE // _CH


@functools.partial(
    pl.kernel,
    out_type=jax.ShapeDtypeStruct((2 * _IDX_ROWS, 1024), jnp.float32),
    mesh=plsc.VectorSubcoreMesh(
        core_axis_name="c", subcore_axis_name="s", num_cores=2, num_subcores=16
    ),
    scratch_types=[
        pltpu.VMEM_SHARED((N_PAD, _LANES), jnp.float32),
        pltpu.VMEM((2, 2 * _CH, 1024), jnp.float32),
        pltpu.VMEM((2, _CH * _IDXW, _LANES), jnp.float32),
        pltpu.VMEM((3, _CH, 2, _IDXW), jnp.int32),
        pltpu.SemaphoreType.DMA,
        pltpu.SemaphoreType.DMA,
        pltpu.SemaphoreType.DMA,
        pltpu.SemaphoreType.DMA,
        pltpu.SemaphoreType.DMA,
    ],
    compiler_params=pltpu.CompilerParams(
        use_tc_tiling_on_sc=False, needs_layout_passes=False
    ),
)
def _sc_dmpnn(
    efeat_hbm, eidx_hbm, out_hbm, neigh, wbuf, tbuf, idx_v,
    sem_i, sem_w, sem_s, sem_g, sem_o,
):
    c = lax.axis_index("c")
    s = lax.axis_index("s")
    iota = lax.iota(jnp.int32, _LANES)
    d0 = iota >> 3
    dcol0 = (iota & 7) * _IDXW

    zrows = N_PAD // 16

    def _zero(i, carry):
        tbuf[0, i] = jnp.zeros((_LANES,), jnp.float32)
        return carry

    lax.fori_loop(0, zrows, _zero, 0)
    pltpu.sync_copy(
        tbuf.at[0, pl.ds(0, zrows)], neigh.at[pl.ds(s * zrows, zrows)]
    )
    plsc.subcore_barrier()

    def _fire_loads(k, rbase, n):
        hi = pltpu.async_copy(
            eidx_hbm.at[pl.ds(rbase, n)],
            idx_v.at[k % 3, pl.ds(0, n)],
            sem_i,
        )
        ha = pltpu.async_copy(
            efeat_hbm.at[pl.ds(rbase, n)],
            wbuf.at[k % 2, pl.ds(0, n)],
            sem_w,
        )
        hb = pltpu.async_copy(
            efeat_hbm.at[pl.ds(_IDX_ROWS + rbase, n)],
            wbuf.at[k % 2, pl.ds(_CH, n)],
            sem_w,
        )
        return (hi, ha, hb)

    def _repack(q, n):
        wq = wbuf.at[q]
        tq = tbuf.at[q]

        def _body(t, carry):
            j = t >> 5
            i = t & 31
            drow = d0 * _CH + j
            dcol = dcol0 + 4 * i
            e = j * _IDXW + 4 * i
            tq[e] = plsc.load_gather(wq, [drow, dcol])
            tq[e + 1] = plsc.load_gather(wq, [drow, dcol + 1])
            tq[e + 2] = plsc.load_gather(wq, [drow, dcol + 2])
            tq[e + 3] = plsc.load_gather(wq, [drow, dcol + 3])
            return carry

        lax.fori_loop(0, n * (_IDXW // 4), _body, 0)

    def _sub_chunk(q, n):
        wq = wbuf.at[q]
        tq = tbuf.at[q]

        def _body(t, carry):
            j = t >> 6
            p = t & 63
            drow = d0 * _CH + j
            dcol = dcol0 + 2 * p
            e = j * _IDXW + 2 * p
            w_e = plsc.load_gather(wq, [drow, dcol])
            w_o = plsc.load_gather(wq, [drow, dcol + 1])
            r_e = tq[e] - w_o
            r_o = tq[e + 1] - w_e
            plsc.store_scatter(wq, [drow, dcol], r_e)
            plsc.store_scatter(wq, [drow, dcol + 1], r_o)
            return carry

        lax.fori_loop(0, n * (_IDXW // 2), _body, 0)

    def _fire_scatters(k, n):
        hs = []
        for j in range(n):
            hs.append(
                pltpu.async_copy(
                    tbuf.at[k % 2, pl.ds(j * _IDXW, _IDXW)],
                    neigh.at[idx_v.at[k % 3, j, 1]],
                    sem_s,
                    add=True,
                )
            )
        return hs

    p1_base = s * _P1_BASE
    loads = _fire_loads(0, p1_base, _CH)
    scats = []
    for k in range(_P1_NCH):
        for h in loads:
            h.wait()
        _repack(k % 2, _CH)
        new_scats = _fire_scatters(k, _CH)
        for h in scats:
            h.wait()
        scats = new_scats
        if k + 1 < _P1_NCH:
            loads = _fire_loads(k + 1, p1_base + (k + 1) * _CH, _CH)
    for h in scats:
        h.wait()

    @pl.when(s < _P1_REM)
    def _p1_rem():
        row = 16 * _P1_BASE + s
        pltpu.sync_copy(eidx_hbm.at[pl.ds(row, 1)], idx_v.at[0, pl.ds(0, 1)])
        pltpu.sync_copy(efeat_hbm.at[pl.ds(row, 1)], wbuf.at[0, pl.ds(0, 1)])
        pltpu.sync_copy(
            efeat_hbm.at[pl.ds(_IDX_ROWS + row, 1)],
            wbuf.at[0, pl.ds(_CH, 1)],
        )
        _repack(0, 1)
        pltpu.sync_copy(
            tbuf.at[0, pl.ds(0, _IDXW)], neigh.at[idx_v.at[0, 0, 1]], add=True
        )

    plsc.subcore_barrier()

    wid = c * 16 + s
    p2_base = wid * _P2_BASE

    def _fire_idx(k, rbase, n):
        return pltpu.async_copy(
            eidx_hbm.at[pl.ds(rbase, n)], idx_v.at[k % 3, pl.ds(0, n)], sem_i
        )

    def _fire_w(k, rbase, n):
        ha = pltpu.async_copy(
            efeat_hbm.at[pl.ds(rbase, n)], wbuf.at[k % 2, pl.ds(0, n)], sem_w
        )
        hb = pltpu.async_copy(
            efeat_hbm.at[pl.ds(_IDX_ROWS + rbase, n)],
            wbuf.at[k % 2, pl.ds(_CH, n)],
            sem_w,
        )
        return (ha, hb)

    def _fire_gathers(k, n):
        hs = []
        for j in range(n):
            hs.append(
                pltpu.async_copy(
                    neigh.at[idx_v.at[k % 3, j, 0]],
                    tbuf.at[k % 2, pl.ds(j * _IDXW, _IDXW)],
                    sem_g,
                )
            )
        return hs

    def _fire_stores(k, rbase, n):
        ha = pltpu.async_copy(
            wbuf.at[k % 2, pl.ds(0, n)], out_hbm.at[pl.ds(rbase, n)], sem_o
        )
        hb = pltpu.async_copy(
            wbuf.at[k % 2, pl.ds(_CH, n)],
            out_hbm.at[pl.ds(_IDX_ROWS + rbase, n)],
            sem_o,
        )
        return (ha, hb)

    idx_h = _fire_idx(0, p2_base, _CH)
    w_h = _fire_w(0, p2_base, _CH)
    st_h = ()
    for k in range(_P2_NCH):
        idx_h.wait()
        g_h = _fire_gathers(k, _CH)
        if k + 1 < _P2_NCH:
            nrb = p2_base + (k + 1) * _CH
            next_idx = _fire_idx(k + 1, nrb, _CH)
            for h in st_h:
                h.wait()
            st_h = ()
            next_w = _fire_w(k + 1, nrb, _CH)
        for h in g_h:
            h.wait()
        for h in w_h:
            h.wait()
        _sub_chunk(k % 2, _CH)
        for h in st_h:
            h.wait()
        st_h = _fire_stores(k, p2_base + k * _CH, _CH)
        if k + 1 < _P2_NCH:
            idx_h = next_idx
            w_h = next_w
    for h in st_h:
        h.wait()

    @pl.when(wid < _P2_REM)
    def _p2_rem():
        row = 32 * _P2_BASE + wid
        pltpu.sync_copy(eidx_hbm.at[pl.ds(row, 1)], idx_v.at[0, pl.ds(0, 1)])
        pltpu.sync_copy(
            neigh.at[idx_v.at[0, 0, 0]], tbuf.at[0, pl.ds(0, _IDXW)]
        )
        pltpu.sync_copy(efeat_hbm.at[pl.ds(row, 1)], wbuf.at[0, pl.ds(0, 1)])
        pltpu.sync_copy(
            efeat_hbm.at[pl.ds(_IDX_ROWS + row, 1)], wbuf.at[0, pl.ds(_CH, 1)]
        )
        _sub_chunk(0, 1)
        pltpu.sync_copy(wbuf.at[0, pl.ds(0, 1)], out_hbm.at[pl.ds(row, 1)])
        pltpu.sync_copy(
            wbuf.at[0, pl.ds(_CH, 1)], out_hbm.at[pl.ds(_IDX_ROWS + row, 1)]
        )


def kernel(nfeat, efeat, edge_index):
    efeat_phys = (
        efeat.T.reshape(2, 8, _IDX_ROWS, _IDXW)
        .transpose(0, 2, 1, 3)
        .reshape(2 * _IDX_ROWS, 1024)
    )
    eidx_phys = edge_index.reshape(2, _IDX_ROWS, _IDXW).transpose(1, 0, 2)
    out_phys = _sc_dmpnn(efeat_phys, eidx_phys)
    return (
        out_phys.reshape(2, _IDX_ROWS, 8, _IDXW)
        .transpose(0, 2, 1, 3)
        .reshape(_LANES, _E)
        .T
    )


# speedup vs baseline: 1.5038x; 1.3832x over previous
import functools

import jax
import jax.numpy as jnp
from jax import lax
from jax.experimental import pallas as pl
from jax.experimental.pallas import tpu as pltpu
from jax.experimental.pallas import tpu_sc as plsc

_LANES = 16
_IDXW = 128
_E = 320000
_N = 10000
_IDX_ROWS = _E // _IDXW
N_PAD = 16 * 626
_P1_BASE = _IDX_ROWS // 16
_P1_REM = _IDX_ROWS - 16 * _P1_BASE
_P2_BASE = _IDX_ROWS // 32
_P2_REM = _IDX_ROWS - 32 * _P2_BASE
_CH = 13
_CHE = _CH * _IDXW
_P1_NCH = _P1_BASE // _CH
_P2_NCH = _P2_BASE // _CH


@functools.partial(
    pl.kernel,
    out_type=jax.ShapeDtypeStruct((_E, _LANES), jnp.float32),
    mesh=plsc.VectorSubcoreMesh(
        core_axis_name="c", subcore_axis_name="s", num_cores=2, num_subcores=16
    ),
    scratch_types=[
        pltpu.VMEM_SHARED((N_PAD, _LANES), jnp.float32),
        pltpu.VMEM((2, _CHE, _LANES), jnp.float32),
        pltpu.VMEM((2, _CHE, _LANES), jnp.float32),
        pltpu.VMEM((3, _CH, 2, _IDXW), jnp.int32),
        pltpu.SemaphoreType.DMA,
        pltpu.SemaphoreType.DMA,
        pltpu.SemaphoreType.DMA,
        pltpu.SemaphoreType.DMA,
        pltpu.SemaphoreType.DMA,
    ],
    compiler_params=pltpu.CompilerParams(
        use_tc_tiling_on_sc=False, needs_layout_passes=False
    ),
)
def _sc_dmpnn(
    efeat_hbm, eidx_hbm, out_hbm, neigh, wbuf, gbuf, idx_v,
    sem_i, sem_w, sem_s, sem_g, sem_o,
):
    c = lax.axis_index("c")
    s = lax.axis_index("s")

    zrows = N_PAD // 16

    def _zero(i, carry):
        wbuf[0, i] = jnp.zeros((_LANES,), jnp.float32)
        return carry

    lax.fori_loop(0, zrows, _zero, 0)
    pltpu.sync_copy(
        wbuf.at[0, pl.ds(0, zrows)], neigh.at[pl.ds(s * zrows, zrows)]
    )
    plsc.subcore_barrier()

    def _fire_idx(k, rbase, n):
        return pltpu.async_copy(
            eidx_hbm.at[pl.ds(rbase, n)], idx_v.at[k % 3, pl.ds(0, n)], sem_i
        )

    def _fire_w(k, rbase, n):
        return pltpu.async_copy(
            efeat_hbm.at[pl.ds(rbase * _IDXW, n * _IDXW)],
            wbuf.at[k % 2, pl.ds(0, n * _IDXW)],
            sem_w,
        )

    def _fire_scatters(k, n):
        hs = []
        for j in range(n):
            hs.append(
                pltpu.async_copy(
                    wbuf.at[k % 2, pl.ds(j * _IDXW, _IDXW)],
                    neigh.at[idx_v.at[k % 3, j, 1]],
                    sem_s,
                    add=True,
                )
            )
        return hs

    p1_base = s * _P1_BASE
    loads = (_fire_idx(0, p1_base, _CH), _fire_w(0, p1_base, _CH))
    scats = []
    for k in range(_P1_NCH):
        for h in loads:
            h.wait()
        new_scats = _fire_scatters(k, _CH)
        for h in scats:
            h.wait()
        scats = new_scats
        if k + 1 < _P1_NCH:
            rb = p1_base + (k + 1) * _CH
            loads = (_fire_idx(k + 1, rb, _CH), _fire_w(k + 1, rb, _CH))
    for h in scats:
        h.wait()

    @pl.when(s < _P1_REM)
    def _p1_rem():
        row = 16 * _P1_BASE + s
        pltpu.sync_copy(eidx_hbm.at[pl.ds(row, 1)], idx_v.at[0, pl.ds(0, 1)])
        pltpu.sync_copy(
            efeat_hbm.at[pl.ds(row * _IDXW, _IDXW)],
            wbuf.at[0, pl.ds(0, _IDXW)],
        )
        pltpu.sync_copy(
            wbuf.at[0, pl.ds(0, _IDXW)], neigh.at[idx_v.at[0, 0, 1]], add=True
        )

    plsc.subcore_barrier()

    wid = c * 16 + s
    p2_base = wid * _P2_BASE

    def _fire_gathers(k, n):
        hs = []
        for j in range(n):
            hs.append(
                pltpu.async_copy(
                    neigh.at[idx_v.at[k % 3, j, 0]],
                    gbuf.at[k % 2, pl.ds(j * _IDXW, _IDXW)],
                    sem_g,
                )
            )
        return hs

    def _fire_store(k, rbase, n):
        return pltpu.async_copy(
            gbuf.at[k % 2, pl.ds(0, n * _IDXW)],
            out_hbm.at[pl.ds(rbase * _IDXW, n * _IDXW)],
            sem_o,
        )

    def _sub_chunk(q, n):
        wq = wbuf.at[q]
        gq = gbuf.at[q]

        def _body(t, carry):
            e = 8 * t
            for u in range(0, 8, 2):
                w_e = wq[e + u]
                w_o = wq[e + u + 1]
                gq[e + u] = gq[e + u] - w_o
                gq[e + u + 1] = gq[e + u + 1] - w_e
            return carry

        lax.fori_loop(0, n * (_IDXW // 8), _body, 0)

    idx_h = _fire_idx(0, p2_base, _CH)
    w_h = _fire_w(0, p2_base, _CH)
    st_h = [None, None]
    for k in range(_P2_NCH):
        idx_h.wait()
        if st_h[k % 2] is not None:
            st_h[k % 2].wait()
        g_h = _fire_gathers(k, _CH)
        if k + 1 < _P2_NCH:
            nrb = p2_base + (k + 1) * _CH
            next_idx = _fire_idx(k + 1, nrb, _CH)
            next_w = _fire_w(k + 1, nrb, _CH)
        for h in g_h:
            h.wait()
        w_h.wait()
        _sub_chunk(k % 2, _CH)
        st_h[k % 2] = _fire_store(k, p2_base + k * _CH, _CH)
        if k + 1 < _P2_NCH:
            idx_h = next_idx
            w_h = next_w
    for h in st_h:
        if h is not None:
            h.wait()

    @pl.when(wid < _P2_REM)
    def _p2_rem():
        row = 32 * _P2_BASE + wid
        pltpu.sync_copy(eidx_hbm.at[pl.ds(row, 1)], idx_v.at[0, pl.ds(0, 1)])
        pltpu.sync_copy(
            neigh.at[idx_v.at[0, 0, 0]], gbuf.at[0, pl.ds(0, _IDXW)]
        )
        pltpu.sync_copy(
            efeat_hbm.at[pl.ds(row * _IDXW, _IDXW)],
            wbuf.at[0, pl.ds(0, _IDXW)],
        )
        _sub_chunk(0, 1)
        pltpu.sync_copy(
            gbuf.at[0, pl.ds(0, _IDXW)], out_hbm.at[pl.ds(row * _IDXW, _IDXW)]
        )


def kernel(nfeat, efeat, edge_index):
    eidx = edge_index.reshape(2, _IDX_ROWS, _IDXW).transpose(1, 0, 2)
    return _sc_dmpnn(efeat, eidx)
